# baseline (speedup 1.0000x reference)
import jax
import jax.numpy as jnp
from jax.experimental import pallas as pl
from jax.experimental.pallas import tpu as pltpu

B, S, D = 1, 2048, 1024
H, DH = 8, 128
BLKQ, BLKK = 128, 64
NQ, NK = S // BLKQ, S // BLKK
KSEL = 4
SCALE = DH ** -0.5
TS = 256

_BF = jnp.bfloat16
_F32 = jnp.float32


def _bdot(a, b):
    return jnp.dot(a.astype(_BF), b.astype(_BF), preferred_element_type=_F32)


def _bdot_t(a, b):
    return jax.lax.dot_general(a.astype(_BF), b.astype(_BF),
                               (((1,), (1,)), ((), ())),
                               preferred_element_type=_F32)


def _roll(x, shift):
    if shift == -1:
        return jnp.concatenate([x[:, 1:], x[:, :1]], axis=1)
    return jnp.concatenate([x[:, -1:], x[:, :-1]], axis=1)


def _norm_rope(x, nw, c, s):
    var = jnp.mean(x * x, axis=-1, keepdims=True)
    r = x * jax.lax.rsqrt(var + 1e-6) * nw
    lane = jax.lax.broadcasted_iota(jnp.int32, r.shape, 1)
    even = (lane % 2) == 0
    shifted = jnp.where(even, -_roll(r, -1), _roll(r, 1))
    return r * c + shifted * s


def _qkv_body(hs_ref, w_ref, b_ref, nqw_ref, nkw_ref, c_ref, s_ref,
              q_ref, k_ref, v_ref):
    x3 = jnp.dot(hs_ref[:], w_ref[:], preferred_element_type=_F32)
    x3 = x3 + b_ref[0]
    c = jnp.tile(c_ref[:], (1, H))
    s = jnp.tile(s_ref[:], (1, H))
    q_ref[:] = _norm_rope(x3[:, :D], nqw_ref[0], c, s)
    k_ref[:] = _norm_rope(x3[:, D:2 * D], nkw_ref[0], c, s)
    v_ref[:] = x3[:, 2 * D:].astype(_BF)


def _stats_body(q_ref, k_ref, v_ref, mq_ref, mk_ref,
                idx_ref, kv_ref, kvt_ref, ks_ref, kst_ref):
    qh = q_ref[:]
    kh = k_ref[:]
    vh = v_ref[:]
    qb = jnp.dot(mq_ref[:], qh, preferred_element_type=_F32,
                 precision=jax.lax.Precision.HIGHEST)
    kb = jnp.dot(mk_ref[:], kh, preferred_element_type=_F32,
                 precision=jax.lax.Precision.HIGHEST)
    sc = _bdot_t(qb, kb)
    iota = jax.lax.broadcasted_iota(jnp.int32, sc.shape, 1)
    cur = sc
    cols = []
    for _ in range(KSEL):
        m = jnp.max(cur, axis=-1, keepdims=True)
        cand = jnp.where(cur >= m, iota, NK)
        amin = jnp.min(cand, axis=-1, keepdims=True)
        cols.append(amin)
        cur = jnp.where(iota == amin, -jnp.inf, cur)
    idx_ref[0] = jnp.concatenate(cols, axis=1)
    phik = jax.nn.softmax(kh, axis=-1)
    kvt = jnp.zeros((DH, DH), _F32)
    kst = jnp.zeros((1, DH), _F32)
    for n in range(NK):
        pb = phik[n * BLKK:(n + 1) * BLKK, :]
        vb = vh[n * BLKK:(n + 1) * BLKK, :]
        kv = jax.lax.dot_general(pb.astype(_BF), vb,
                                 (((0,), (0,)), ((), ())),
                                 preferred_element_type=_F32)
        kvb = kv.astype(_BF)
        ksn = jnp.sum(pb, axis=0, keepdims=True)
        kv_ref[0, n] = kvb
        ks_ref[0, n] = ksn[0]
        kvt = kvt + kvb.astype(_F32)
        kst = kst + ksn
    kvt_ref[0] = kvt
    kst_ref[0] = kst


QB = 4


def _attn_body(idx_sref, q_ref, k_ref, v_ref, kv_ref, kvt_ref,
               ks_ref, kst_ref, wo_ref, bo_ref, out_ref):
    h = pl.program_id(0)
    qg = pl.program_id(1)
    kvt = kvt_ref[0]
    kst = kst_ref[0]
    wo_h = wo_ref[h]
    parts = []
    for qq in range(QB):
        qi = qg * QB + qq
        q = q_ref[qq * BLKQ:(qq + 1) * BLKQ, :]
        kvq = kvt
        ksq = kst
        ksl = []
        vsl = []
        for j in range(KSEL):
            bi = idx_sref[h, qi, j]
            ksl.append(k_ref[pl.ds(bi * BLKK, BLKK), :])
            vsl.append(v_ref[pl.ds(bi * BLKK, BLKK), :])
            kvq = kvq - kv_ref[0, pl.ds(bi, 1), :, :][0].astype(_F32)
            ksq = ksq - ks_ref[0, pl.ds(bi, 1), :]
        ksel = jnp.concatenate(ksl, axis=0)
        vsel = jnp.concatenate(vsl, axis=0)
        logits = _bdot_t(q, ksel) * SCALE
        m = jnp.max(logits, axis=-1, keepdims=True)
        p = jnp.exp(logits - m)
        attn = p / jnp.sum(p, axis=-1, keepdims=True)
        o_sp = jnp.dot(attn.astype(_BF), vsel, preferred_element_type=_F32)
        phiq = jax.nn.softmax(q, axis=-1)
        num = _bdot(phiq, kvq)
        den = jnp.sum(phiq * ksq, axis=-1, keepdims=True) + 1e-6
        ob = (o_sp + num / den).astype(_BF)
        parts.append(jnp.dot(ob, wo_h, preferred_element_type=_F32))
    contrib = jnp.concatenate(parts, axis=0)
    rows = pl.ds(qg * QB * BLKQ, QB * BLKQ)

    @pl.when(h == 0)
    def _():
        out_ref[rows, :] = contrib + bo_ref[0]

    @pl.when(h > 0)
    def _():
        out_ref[rows, :] = out_ref[rows, :] + contrib


def kernel(hidden_states, freqs_cos, freqs_sin, Wq, bq, Wk, bk, Wv, bv,
           Wo, bo, norm_q_w, norm_k_w):
    hs2 = hidden_states.reshape(S, D).astype(_BF)
    cos_e = freqs_cos.reshape(S, DH)[:, 0::2]
    sin_o = freqs_sin.reshape(S, DH)[:, 1::2]
    chead = jnp.repeat(cos_e, 2, axis=1)
    shead = jnp.repeat(sin_o, 2, axis=1)
    wcat = jnp.concatenate([Wq, Wk, Wv], axis=1).astype(_BF)
    bcat = jnp.concatenate([bq, bk, bv]).reshape(1, 3 * D)
    nqw = norm_q_w.reshape(1, D)
    nkw = norm_k_w.reshape(1, D)
    mq = jnp.repeat(jnp.eye(NQ, dtype=_F32), BLKQ, axis=1) / BLKQ
    mk = jnp.repeat(jnp.eye(NK, dtype=_F32), BLKK, axis=1) / BLKK

    q2, k2, v2 = pl.pallas_call(
        _qkv_body,
        grid=(S // TS,),
        in_specs=[
            pl.BlockSpec((TS, D), lambda i: (i, 0)),
            pl.BlockSpec((D, 3 * D), lambda i: (0, 0)),
            pl.BlockSpec((1, 3 * D), lambda i: (0, 0)),
            pl.BlockSpec((1, D), lambda i: (0, 0)),
            pl.BlockSpec((1, D), lambda i: (0, 0)),
            pl.BlockSpec((TS, DH), lambda i: (i, 0)),
            pl.BlockSpec((TS, DH), lambda i: (i, 0)),
        ],
        out_specs=[
            pl.BlockSpec((TS, D), lambda i: (i, 0)),
            pl.BlockSpec((TS, D), lambda i: (i, 0)),
            pl.BlockSpec((TS, D), lambda i: (i, 0)),
        ],
        out_shape=[
            jax.ShapeDtypeStruct((S, D), _F32),
            jax.ShapeDtypeStruct((S, D), _F32),
            jax.ShapeDtypeStruct((S, D), _BF),
        ],
    )(hs2, wcat, bcat, nqw, nkw, chead, shead)

    idx, kvb, kvt, ksb, kst = pl.pallas_call(
        _stats_body,
        grid=(H,),
        in_specs=[
            pl.BlockSpec((S, DH), lambda h: (0, h)),
            pl.BlockSpec((S, DH), lambda h: (0, h)),
            pl.BlockSpec((S, DH), lambda h: (0, h)),
            pl.BlockSpec((NQ, S), lambda h: (0, 0)),
            pl.BlockSpec((NK, S), lambda h: (0, 0)),
        ],
        out_specs=[
            pl.BlockSpec((1, NQ, KSEL), lambda h: (h, 0, 0)),
            pl.BlockSpec((1, NK, DH, DH), lambda h: (h, 0, 0, 0)),
            pl.BlockSpec((1, DH, DH), lambda h: (h, 0, 0)),
            pl.BlockSpec((1, NK, DH), lambda h: (h, 0, 0)),
            pl.BlockSpec((1, 1, DH), lambda h: (h, 0, 0)),
        ],
        out_shape=[
            jax.ShapeDtypeStruct((H, NQ, KSEL), jnp.int32),
            jax.ShapeDtypeStruct((H, NK, DH, DH), _BF),
            jax.ShapeDtypeStruct((H, DH, DH), _F32),
            jax.ShapeDtypeStruct((H, NK, DH), _F32),
            jax.ShapeDtypeStruct((H, 1, DH), _F32),
        ],
    )(q2, k2, v2, mq, mk)

    wo_r = Wo.reshape(H, DH, D).astype(_BF)
    out = pl.pallas_call(
        _attn_body,
        grid_spec=pltpu.PrefetchScalarGridSpec(
            num_scalar_prefetch=1,
            grid=(H, NQ // QB),
            in_specs=[
                pl.BlockSpec((QB * BLKQ, DH), lambda h, qg, *_: (qg, h)),
                pl.BlockSpec((S, DH), lambda h, qg, *_: (0, h)),
                pl.BlockSpec((S, DH), lambda h, qg, *_: (0, h)),
                pl.BlockSpec((1, NK, DH, DH), lambda h, qg, *_: (h, 0, 0, 0)),
                pl.BlockSpec((1, DH, DH), lambda h, qg, *_: (h, 0, 0)),
                pl.BlockSpec((1, NK, DH), lambda h, qg, *_: (h, 0, 0)),
                pl.BlockSpec((1, 1, DH), lambda h, qg, *_: (h, 0, 0)),
                pl.BlockSpec((H, DH, D), lambda h, qg, *_: (0, 0, 0)),
                pl.BlockSpec((1, D), lambda h, qg, *_: (0, 0)),
            ],
            out_specs=pl.BlockSpec((S, D), lambda h, qg, *_: (0, 0)),
        ),
        out_shape=jax.ShapeDtypeStruct((S, D), _F32),
    )(idx, q2, k2, v2, kvb, kvt, ksb, kst, wo_r, bo.reshape(1, D))

    return out.reshape(B, S, D)


# speedup vs baseline: 1.2275x; 1.2275x over previous
import jax
import jax.numpy as jnp
from jax.experimental import pallas as pl
from jax.experimental.pallas import tpu as pltpu

B, S, D = 1, 2048, 1024
H, DH = 8, 128
BLKQ, BLKK = 128, 64
NQ, NK = S // BLKQ, S // BLKK
KSEL = 4
SCALE = DH ** -0.5
TS = 256
QB = 4

_BF = jnp.bfloat16
_F32 = jnp.float32


def _bdot(a, b):
    return jnp.dot(a.astype(_BF), b.astype(_BF), preferred_element_type=_F32)


def _bdot_t(a, b):
    return jax.lax.dot_general(a.astype(_BF), b.astype(_BF),
                               (((1,), (1,)), ((), ())),
                               preferred_element_type=_F32)


def _roll(x, shift):
    if shift == -1:
        return jnp.concatenate([x[:, 1:], x[:, :1]], axis=1)
    return jnp.concatenate([x[:, -1:], x[:, :-1]], axis=1)


def _norm_rope(x, nw, c, s):
    var = jnp.mean(x * x, axis=-1, keepdims=True)
    r = x * jax.lax.rsqrt(var + 1e-6) * nw
    lane = jax.lax.broadcasted_iota(jnp.int32, r.shape, 1)
    even = (lane % 2) == 0
    shifted = jnp.where(even, -_roll(r, -1), _roll(r, 1))
    return r * c + shifted * s


def _qkv_body(hs_ref, w_ref, b_ref, nqw_ref, nkw_ref, c_ref, s_ref,
              m2_ref, m4_ref, q_ref, k_ref, v_ref, qb_ref, kb_ref):
    x3 = jnp.dot(hs_ref[:], w_ref[:], preferred_element_type=_F32)
    x3 = x3 + b_ref[0]
    c = jnp.tile(c_ref[:], (1, H))
    s = jnp.tile(s_ref[:], (1, H))
    qr = _norm_rope(x3[:, :D], nqw_ref[0], c, s)
    kr = _norm_rope(x3[:, D:2 * D], nkw_ref[0], c, s)
    q_ref[:] = qr
    k_ref[:] = kr
    v_ref[:] = x3[:, 2 * D:].astype(_BF)
    qb_ref[0] = jnp.dot(m2_ref[:], qr, preferred_element_type=_F32,
                        precision=jax.lax.Precision.HIGHEST)
    kb_ref[0] = jnp.dot(m4_ref[:], kr, preferred_element_type=_F32,
                        precision=jax.lax.Precision.HIGHEST)


def _score_body(qb_ref, kb_ref, idx_ref):
    for h in range(H):
        qb = qb_ref[:, h * DH:(h + 1) * DH]
        kb = kb_ref[:, h * DH:(h + 1) * DH]
        sc = _bdot_t(qb, kb)
        iota = jax.lax.broadcasted_iota(jnp.int32, sc.shape, 1)
        cur = sc
        cols = []
        for _ in range(KSEL):
            m = jnp.max(cur, axis=-1, keepdims=True)
            cand = jnp.where(cur >= m, iota, NK)
            amin = jnp.min(cand, axis=-1, keepdims=True)
            cols.append(amin)
            cur = jnp.where(iota == amin, -jnp.inf, cur)
        idx_ref[h] = jnp.concatenate(cols, axis=1)


def _attn_body(idx_sref, q_ref, k_ref, v_ref, o_ref,
               kv_scr, ks_scr, kvt_scr, kst_scr):
    h = pl.program_id(0)
    qg = pl.program_id(1)

    @pl.when(qg == 0)
    def _():
        kh = k_ref[:]
        vh = v_ref[:]
        phik = jax.nn.softmax(kh, axis=-1)
        kvt = jnp.zeros((DH, DH), _F32)
        kst = jnp.zeros((1, DH), _F32)
        for n in range(NK):
            pb = phik[n * BLKK:(n + 1) * BLKK, :]
            vb = vh[n * BLKK:(n + 1) * BLKK, :]
            kv = jax.lax.dot_general(pb.astype(_BF), vb,
                                     (((0,), (0,)), ((), ())),
                                     preferred_element_type=_F32)
            kvb = kv.astype(_BF)
            ksn = jnp.sum(pb, axis=0, keepdims=True)
            kv_scr[n] = kvb
            ks_scr[n] = ksn
            kvt = kvt + kvb.astype(_F32)
            kst = kst + ksn
        kvt_scr[:] = kvt
        kst_scr[:] = kst

    @pl.when(qg > 0)
    def _():
        kvt = kvt_scr[:]
        kst = kst_scr[:]
        for qq in range(QB):
            qi = (qg - 1) * QB + qq
            q = q_ref[qq * BLKQ:(qq + 1) * BLKQ, :]
            kvq = kvt
            ksq = kst
            ksl = []
            vsl = []
            for j in range(KSEL):
                bi = idx_sref[h, qi, j]
                ksl.append(k_ref[pl.ds(bi * BLKK, BLKK), :])
                vsl.append(v_ref[pl.ds(bi * BLKK, BLKK), :])
                kvq = kvq - kv_scr[pl.ds(bi, 1), :, :][0].astype(_F32)
                ksq = ksq - ks_scr[pl.ds(bi, 1), 0, :]
            ksel = jnp.concatenate(ksl, axis=0)
            vsel = jnp.concatenate(vsl, axis=0)
            logits = _bdot_t(q, ksel) * SCALE
            m = jnp.max(logits, axis=-1, keepdims=True)
            p = jnp.exp(logits - m)
            attn = p / jnp.sum(p, axis=-1, keepdims=True)
            o_sp = jnp.dot(attn.astype(_BF), vsel, preferred_element_type=_F32)
            phiq = jax.nn.softmax(q, axis=-1)
            num = _bdot(phiq, kvq)
            den = jnp.sum(phiq * ksq, axis=-1, keepdims=True) + 1e-6
            o_ref[0, qq * BLKQ:(qq + 1) * BLKQ, :] = \
                (o_sp + num / den).astype(_BF)


def _oproj_body(o_ref, wo_ref, bo_ref, out_ref):
    acc = jnp.zeros((TS, D), _F32) + bo_ref[0, 0]
    for h in range(H):
        acc = acc + jnp.dot(o_ref[h], wo_ref[h], preferred_element_type=_F32)
    out_ref[:] = acc


def kernel(hidden_states, freqs_cos, freqs_sin, Wq, bq, Wk, bk, Wv, bv,
           Wo, bo, norm_q_w, norm_k_w):
    hs2 = hidden_states.reshape(S, D).astype(_BF)
    cos_e = freqs_cos.reshape(S, DH)[:, 0::2]
    sin_o = freqs_sin.reshape(S, DH)[:, 1::2]
    chead = jnp.repeat(cos_e, 2, axis=1)
    shead = jnp.repeat(sin_o, 2, axis=1)
    wcat = jnp.concatenate([Wq, Wk, Wv], axis=1).astype(_BF)
    bcat = jnp.concatenate([bq, bk, bv]).reshape(1, 3 * D)
    nqw = norm_q_w.reshape(1, D)
    nkw = norm_k_w.reshape(1, D)
    nqt = TS // BLKQ
    nkt = TS // BLKK
    m2 = jnp.repeat(jnp.eye(nqt, dtype=_F32), BLKQ, axis=1) / BLKQ
    m4 = jnp.repeat(jnp.eye(nkt, dtype=_F32), BLKK, axis=1) / BLKK

    q2, k2, v2, qb3, kb3 = pl.pallas_call(
        _qkv_body,
        grid=(S // TS,),
        in_specs=[
            pl.BlockSpec((TS, D), lambda i: (i, 0)),
            pl.BlockSpec((D, 3 * D), lambda i: (0, 0)),
            pl.BlockSpec((1, 3 * D), lambda i: (0, 0)),
            pl.BlockSpec((1, D), lambda i: (0, 0)),
            pl.BlockSpec((1, D), lambda i: (0, 0)),
            pl.BlockSpec((TS, DH), lambda i: (i, 0)),
            pl.BlockSpec((TS, DH), lambda i: (i, 0)),
            pl.BlockSpec((nqt, TS), lambda i: (0, 0)),
            pl.BlockSpec((nkt, TS), lambda i: (0, 0)),
        ],
        out_specs=[
            pl.BlockSpec((TS, D), lambda i: (i, 0)),
            pl.BlockSpec((TS, D), lambda i: (i, 0)),
            pl.BlockSpec((TS, D), lambda i: (i, 0)),
            pl.BlockSpec((1, nqt, D), lambda i: (i, 0, 0)),
            pl.BlockSpec((1, nkt, D), lambda i: (i, 0, 0)),
        ],
        out_shape=[
            jax.ShapeDtypeStruct((S, D), _F32),
            jax.ShapeDtypeStruct((S, D), _F32),
            jax.ShapeDtypeStruct((S, D), _BF),
            jax.ShapeDtypeStruct((S // TS, nqt, D), _F32),
            jax.ShapeDtypeStruct((S // TS, nkt, D), _F32),
        ],
    )(hs2, wcat, bcat, nqw, nkw, chead, shead, m2, m4)
    qb_all = qb3.reshape(NQ, D)
    kb_all = kb3.reshape(NK, D)

    idx = pl.pallas_call(
        _score_body,
        grid=(1,),
        in_specs=[
            pl.BlockSpec((NQ, D), lambda i: (0, 0)),
            pl.BlockSpec((NK, D), lambda i: (0, 0)),
        ],
        out_specs=pl.BlockSpec((H, NQ, KSEL), lambda i: (0, 0, 0)),
        out_shape=jax.ShapeDtypeStruct((H, NQ, KSEL), jnp.int32),
    )(qb_all, kb_all)

    o_heads = pl.pallas_call(
        _attn_body,
        grid_spec=pltpu.PrefetchScalarGridSpec(
            num_scalar_prefetch=1,
            grid=(H, 1 + NQ // QB),
            in_specs=[
                pl.BlockSpec((QB * BLKQ, DH),
                             lambda h, qg, *_: (jnp.maximum(qg - 1, 0), h)),
                pl.BlockSpec((S, DH), lambda h, qg, *_: (0, h)),
                pl.BlockSpec((S, DH), lambda h, qg, *_: (0, h)),
            ],
            out_specs=pl.BlockSpec(
                (1, QB * BLKQ, DH),
                lambda h, qg, *_: (h, jnp.maximum(qg - 1, 0), 0)),
            scratch_shapes=[
                pltpu.VMEM((NK, DH, DH), _BF),
                pltpu.VMEM((NK, 1, DH), _F32),
                pltpu.VMEM((DH, DH), _F32),
                pltpu.VMEM((1, DH), _F32),
            ],
        ),
        out_shape=jax.ShapeDtypeStruct((H, S, DH), _BF),
    )(idx, q2, k2, v2)

    wo_r = Wo.reshape(H, DH, D).astype(_BF)
    out = pl.pallas_call(
        _oproj_body,
        grid=(S // TS,),
        in_specs=[
            pl.BlockSpec((H, TS, DH), lambda i: (0, i, 0)),
            pl.BlockSpec((H, DH, D), lambda i: (0, 0, 0)),
            pl.BlockSpec((1, 1, D), lambda i: (0, 0, 0)),
        ],
        out_specs=pl.BlockSpec((TS, D), lambda i: (i, 0)),
        out_shape=jax.ShapeDtypeStruct((S, D), _F32),
    )(o_heads, wo_r, bo.reshape(1, 1, D))

    return out.reshape(B, S, D)


# speedup vs baseline: 1.2771x; 1.0405x over previous
import jax
import jax.numpy as jnp
from jax.experimental import pallas as pl
from jax.experimental.pallas import tpu as pltpu

B, S, D = 1, 2048, 1024
H, DH = 8, 128
BLKQ, BLKK = 128, 64
NQ, NK = S // BLKQ, S // BLKK
KSEL = 4
SCALE = DH ** -0.5
TS = 256
QB = 8

_BF = jnp.bfloat16
_F32 = jnp.float32


def _bdot(a, b):
    return jnp.dot(a.astype(_BF), b.astype(_BF), preferred_element_type=_F32)


def _bdot_t(a, b):
    return jax.lax.dot_general(a.astype(_BF), b.astype(_BF),
                               (((1,), (1,)), ((), ())),
                               preferred_element_type=_F32)


def _roll(x, shift):
    if shift == -1:
        return jnp.concatenate([x[:, 1:], x[:, :1]], axis=1)
    return jnp.concatenate([x[:, -1:], x[:, :-1]], axis=1)


def _norm_rope(x, nw, c, s):
    var = jnp.mean(x * x, axis=-1, keepdims=True)
    r = x * jax.lax.rsqrt(var + 1e-6) * nw
    lane = jax.lax.broadcasted_iota(jnp.int32, r.shape, 1)
    even = (lane % 2) == 0
    shifted = jnp.where(even, -_roll(r, -1), _roll(r, 1))
    return r * c + shifted * s


def _qkv_body(hs_ref, w_ref, b_ref, nqw_ref, nkw_ref, c_ref, s_ref,
              m2_ref, m4_ref, q_ref, k_ref, v_ref, qb_ref, kb_ref):
    x3 = jnp.dot(hs_ref[:], w_ref[:], preferred_element_type=_F32)
    x3 = x3 + b_ref[0]
    c = jnp.tile(c_ref[:], (1, H))
    s = jnp.tile(s_ref[:], (1, H))
    qr = _norm_rope(x3[:, :D], nqw_ref[0], c, s)
    kr = _norm_rope(x3[:, D:2 * D], nkw_ref[0], c, s)
    q_ref[:] = qr
    k_ref[:] = kr
    v_ref[:] = x3[:, 2 * D:].astype(_BF)
    qb_ref[0] = jnp.dot(m2_ref[:], qr, preferred_element_type=_F32,
                        precision=jax.lax.Precision.HIGHEST)
    kb_ref[0] = jnp.dot(m4_ref[:], kr, preferred_element_type=_F32,
                        precision=jax.lax.Precision.HIGHEST)


def _score_body(qb_ref, kb_ref, idx_ref):
    for h in range(H):
        qb = qb_ref[:, h * DH:(h + 1) * DH]
        kb = kb_ref[:, h * DH:(h + 1) * DH]
        sc = _bdot_t(qb, kb)
        iota = jax.lax.broadcasted_iota(jnp.int32, sc.shape, 1)
        cur = sc
        cols = []
        for _ in range(KSEL):
            m = jnp.max(cur, axis=-1, keepdims=True)
            cand = jnp.where(cur >= m, iota, NK)
            amin = jnp.min(cand, axis=-1, keepdims=True)
            cols.append(amin)
            cur = jnp.where(iota == amin, -jnp.inf, cur)
        idx_ref[h] = jnp.concatenate(cols, axis=1)


def _attn_body(idx_sref, q_ref, k_ref, v_ref, o_ref,
               kv_scr, ks_scr, kvt_scr, kst_scr):
    h = pl.program_id(0)
    qg = pl.program_id(1)

    @pl.when(qg == 0)
    def _():
        kh = k_ref[:]
        vh = v_ref[:]
        phik = jax.nn.softmax(kh, axis=-1)
        kvt = jnp.zeros((DH, DH), _F32)
        kst = jnp.zeros((1, DH), _F32)
        for n in range(NK):
            pb = phik[n * BLKK:(n + 1) * BLKK, :]
            vb = vh[n * BLKK:(n + 1) * BLKK, :]
            kv = jax.lax.dot_general(pb.astype(_BF), vb,
                                     (((0,), (0,)), ((), ())),
                                     preferred_element_type=_F32)
            kvb = kv.astype(_BF)
            ksn = jnp.sum(pb, axis=0, keepdims=True)
            kv_scr[n] = kvb
            ks_scr[n] = ksn
            kvt = kvt + kvb.astype(_F32)
            kst = kst + ksn
        kvt_scr[:] = kvt
        kst_scr[:] = kst

    @pl.when(qg > 0)
    def _():
        kvt = kvt_scr[:]
        kst = kst_scr[:]
        for qq in range(QB):
            qi = (qg - 1) * QB + qq
            q = q_ref[qq * BLKQ:(qq + 1) * BLKQ, :]
            kvq = kvt
            ksq = kst
            ksl = []
            vsl = []
            for j in range(KSEL):
                bi = idx_sref[h, qi, j]
                ksl.append(k_ref[pl.ds(bi * BLKK, BLKK), :])
                vsl.append(v_ref[pl.ds(bi * BLKK, BLKK), :])
                kvq = kvq - kv_scr[pl.ds(bi, 1), :, :][0].astype(_F32)
                ksq = ksq - ks_scr[pl.ds(bi, 1), 0, :]
            ksel = jnp.concatenate(ksl, axis=0)
            vsel = jnp.concatenate(vsl, axis=0)
            logits = _bdot_t(q, ksel) * SCALE
            m = jnp.max(logits, axis=-1, keepdims=True)
            p = jnp.exp(logits - m)
            attn = p / jnp.sum(p, axis=-1, keepdims=True)
            o_sp = jnp.dot(attn.astype(_BF), vsel, preferred_element_type=_F32)
            phiq = jax.nn.softmax(q, axis=-1)
            num = _bdot(phiq, kvq)
            den = jnp.sum(phiq * ksq, axis=-1, keepdims=True) + 1e-6
            o_ref[0, qq * BLKQ:(qq + 1) * BLKQ, :] = \
                (o_sp + num / den).astype(_BF)


def _oproj_body(o_ref, wo_ref, bo_ref, out_ref):
    acc = jnp.zeros((TS, D), _F32) + bo_ref[0, 0]
    for h in range(H):
        acc = acc + jnp.dot(o_ref[h], wo_ref[h], preferred_element_type=_F32)
    out_ref[:] = acc


def kernel(hidden_states, freqs_cos, freqs_sin, Wq, bq, Wk, bk, Wv, bv,
           Wo, bo, norm_q_w, norm_k_w):
    hs2 = hidden_states.reshape(S, D).astype(_BF)
    cos_e = freqs_cos.reshape(S, DH)[:, 0::2]
    sin_o = freqs_sin.reshape(S, DH)[:, 1::2]
    chead = jnp.repeat(cos_e, 2, axis=1)
    shead = jnp.repeat(sin_o, 2, axis=1)
    wcat = jnp.concatenate([Wq, Wk, Wv], axis=1).astype(_BF)
    bcat = jnp.concatenate([bq, bk, bv]).reshape(1, 3 * D)
    nqw = norm_q_w.reshape(1, D)
    nkw = norm_k_w.reshape(1, D)
    nqt = TS // BLKQ
    nkt = TS // BLKK
    m2 = jnp.repeat(jnp.eye(nqt, dtype=_F32), BLKQ, axis=1) / BLKQ
    m4 = jnp.repeat(jnp.eye(nkt, dtype=_F32), BLKK, axis=1) / BLKK

    q2, k2, v2, qb3, kb3 = pl.pallas_call(
        _qkv_body,
        grid=(S // TS,),
        in_specs=[
            pl.BlockSpec((TS, D), lambda i: (i, 0)),
            pl.BlockSpec((D, 3 * D), lambda i: (0, 0)),
            pl.BlockSpec((1, 3 * D), lambda i: (0, 0)),
            pl.BlockSpec((1, D), lambda i: (0, 0)),
            pl.BlockSpec((1, D), lambda i: (0, 0)),
            pl.BlockSpec((TS, DH), lambda i: (i, 0)),
            pl.BlockSpec((TS, DH), lambda i: (i, 0)),
            pl.BlockSpec((nqt, TS), lambda i: (0, 0)),
            pl.BlockSpec((nkt, TS), lambda i: (0, 0)),
        ],
        out_specs=[
            pl.BlockSpec((TS, D), lambda i: (i, 0)),
            pl.BlockSpec((TS, D), lambda i: (i, 0)),
            pl.BlockSpec((TS, D), lambda i: (i, 0)),
            pl.BlockSpec((1, nqt, D), lambda i: (i, 0, 0)),
            pl.BlockSpec((1, nkt, D), lambda i: (i, 0, 0)),
        ],
        out_shape=[
            jax.ShapeDtypeStruct((S, D), _F32),
            jax.ShapeDtypeStruct((S, D), _F32),
            jax.ShapeDtypeStruct((S, D), _BF),
            jax.ShapeDtypeStruct((S // TS, nqt, D), _F32),
            jax.ShapeDtypeStruct((S // TS, nkt, D), _F32),
        ],
    )(hs2, wcat, bcat, nqw, nkw, chead, shead, m2, m4)
    qb_all = qb3.reshape(NQ, D)
    kb_all = kb3.reshape(NK, D)

    idx = pl.pallas_call(
        _score_body,
        grid=(1,),
        in_specs=[
            pl.BlockSpec((NQ, D), lambda i: (0, 0)),
            pl.BlockSpec((NK, D), lambda i: (0, 0)),
        ],
        out_specs=pl.BlockSpec((H, NQ, KSEL), lambda i: (0, 0, 0)),
        out_shape=jax.ShapeDtypeStruct((H, NQ, KSEL), jnp.int32),
    )(qb_all, kb_all)

    o_heads = pl.pallas_call(
        _attn_body,
        grid_spec=pltpu.PrefetchScalarGridSpec(
            num_scalar_prefetch=1,
            grid=(H, 1 + NQ // QB),
            in_specs=[
                pl.BlockSpec((QB * BLKQ, DH),
                             lambda h, qg, *_: (jnp.maximum(qg - 1, 0), h)),
                pl.BlockSpec((S, DH), lambda h, qg, *_: (0, h)),
                pl.BlockSpec((S, DH), lambda h, qg, *_: (0, h)),
            ],
            out_specs=pl.BlockSpec(
                (1, QB * BLKQ, DH),
                lambda h, qg, *_: (h, jnp.maximum(qg - 1, 0), 0)),
            scratch_shapes=[
                pltpu.VMEM((NK, DH, DH), _BF),
                pltpu.VMEM((NK, 1, DH), _F32),
                pltpu.VMEM((DH, DH), _F32),
                pltpu.VMEM((1, DH), _F32),
            ],
        ),
        out_shape=jax.ShapeDtypeStruct((H, S, DH), _BF),
    )(idx, q2, k2, v2)

    wo_r = Wo.reshape(H, DH, D).astype(_BF)
    out = pl.pallas_call(
        _oproj_body,
        grid=(S // TS,),
        in_specs=[
            pl.BlockSpec((H, TS, DH), lambda i: (0, i, 0)),
            pl.BlockSpec((H, DH, D), lambda i: (0, 0, 0)),
            pl.BlockSpec((1, 1, D), lambda i: (0, 0, 0)),
        ],
        out_specs=pl.BlockSpec((TS, D), lambda i: (i, 0)),
        out_shape=jax.ShapeDtypeStruct((S, D), _F32),
    )(o_heads, wo_r, bo.reshape(1, 1, D))

    return out.reshape(B, S, D)


# speedup vs baseline: 1.2964x; 1.0151x over previous
import jax
import jax.numpy as jnp
from jax.experimental import pallas as pl
from jax.experimental.pallas import tpu as pltpu

B, S, D = 1, 2048, 1024
H, DH = 8, 128
BLKQ, BLKK = 128, 64
NQ, NK = S // BLKQ, S // BLKK
KSEL = 4
SCALE = DH ** -0.5
TS = 256
QB = 16

_BF = jnp.bfloat16
_F32 = jnp.float32


def _bdot(a, b):
    return jnp.dot(a.astype(_BF), b.astype(_BF), preferred_element_type=_F32)


def _bdot_t(a, b):
    return jax.lax.dot_general(a.astype(_BF), b.astype(_BF),
                               (((1,), (1,)), ((), ())),
                               preferred_element_type=_F32)


def _roll(x, shift):
    if shift == -1:
        return jnp.concatenate([x[:, 1:], x[:, :1]], axis=1)
    return jnp.concatenate([x[:, -1:], x[:, :-1]], axis=1)


def _norm_rope(x, nw, c, s):
    var = jnp.mean(x * x, axis=-1, keepdims=True)
    r = x * jax.lax.rsqrt(var + 1e-6) * nw
    lane = jax.lax.broadcasted_iota(jnp.int32, r.shape, 1)
    even = (lane % 2) == 0
    shifted = jnp.where(even, -_roll(r, -1), _roll(r, 1))
    return r * c + shifted * s


def _qkv_body(hs_ref, w_ref, b_ref, nqw_ref, nkw_ref, c_ref, s_ref,
              m2_ref, m4_ref, q_ref, k_ref, v_ref, qb_ref, kb_ref):
    x3 = jnp.dot(hs_ref[:], w_ref[:], preferred_element_type=_F32)
    x3 = x3 + b_ref[0]
    c = jnp.tile(c_ref[:], (1, H))
    s = jnp.tile(s_ref[:], (1, H))
    qr = _norm_rope(x3[:, :D], nqw_ref[0], c, s)
    kr = _norm_rope(x3[:, D:2 * D], nkw_ref[0], c, s)
    q_ref[:] = qr
    k_ref[:] = kr
    v_ref[:] = x3[:, 2 * D:].astype(_BF)
    qb_ref[0] = jnp.dot(m2_ref[:], qr, preferred_element_type=_F32,
                        precision=jax.lax.Precision.HIGHEST)
    kb_ref[0] = jnp.dot(m4_ref[:], kr, preferred_element_type=_F32,
                        precision=jax.lax.Precision.HIGHEST)


def _score_body(qb_ref, kb_ref, idx_ref):
    for h in range(H):
        qb = qb_ref[:, h * DH:(h + 1) * DH]
        kb = kb_ref[:, h * DH:(h + 1) * DH]
        sc = _bdot_t(qb, kb)
        iota = jax.lax.broadcasted_iota(jnp.int32, sc.shape, 1)
        cur = sc
        cols = []
        for _ in range(KSEL):
            m = jnp.max(cur, axis=-1, keepdims=True)
            cand = jnp.where(cur >= m, iota, NK)
            amin = jnp.min(cand, axis=-1, keepdims=True)
            cols.append(amin)
            cur = jnp.where(iota == amin, -jnp.inf, cur)
        idx_ref[h] = jnp.concatenate(cols, axis=1)


def _attn_body(idx_sref, q_ref, k_ref, v_ref, o_ref,
               kv_scr, ks_scr, kvt_scr, kst_scr):
    h = pl.program_id(0)
    qg = pl.program_id(1)

    @pl.when(qg == 0)
    def _():
        kh = k_ref[:]
        vh = v_ref[:]
        phik = jax.nn.softmax(kh, axis=-1)
        kvt = jnp.zeros((DH, DH), _F32)
        kst = jnp.zeros((1, DH), _F32)
        for n in range(NK):
            pb = phik[n * BLKK:(n + 1) * BLKK, :]
            vb = vh[n * BLKK:(n + 1) * BLKK, :]
            kv = jax.lax.dot_general(pb.astype(_BF), vb,
                                     (((0,), (0,)), ((), ())),
                                     preferred_element_type=_F32)
            kvb = kv.astype(_BF)
            ksn = jnp.sum(pb, axis=0, keepdims=True)
            kv_scr[n] = kvb
            ks_scr[n] = ksn
            kvt = kvt + kvb.astype(_F32)
            kst = kst + ksn
        kvt_scr[:] = kvt
        kst_scr[:] = kst

    @pl.when(qg > 0)
    def _():
        kvt = kvt_scr[:]
        kst = kst_scr[:]
        for qq in range(QB):
            qi = (qg - 1) * QB + qq
            q = q_ref[qq * BLKQ:(qq + 1) * BLKQ, :]
            kvq = kvt
            ksq = kst
            ksl = []
            vsl = []
            for j in range(KSEL):
                bi = idx_sref[h, qi, j]
                ksl.append(k_ref[pl.ds(bi * BLKK, BLKK), :])
                vsl.append(v_ref[pl.ds(bi * BLKK, BLKK), :])
                kvq = kvq - kv_scr[pl.ds(bi, 1), :, :][0].astype(_F32)
                ksq = ksq - ks_scr[pl.ds(bi, 1), 0, :]
            ksel = jnp.concatenate(ksl, axis=0)
            vsel = jnp.concatenate(vsl, axis=0)
            logits = _bdot_t(q, ksel) * SCALE
            m = jnp.max(logits, axis=-1, keepdims=True)
            p = jnp.exp(logits - m)
            attn = p / jnp.sum(p, axis=-1, keepdims=True)
            o_sp = jnp.dot(attn.astype(_BF), vsel, preferred_element_type=_F32)
            phiq = jax.nn.softmax(q, axis=-1)
            num = _bdot(phiq, kvq)
            den = jnp.sum(phiq * ksq, axis=-1, keepdims=True) + 1e-6
            o_ref[0, qq * BLKQ:(qq + 1) * BLKQ, :] = \
                (o_sp + num / den).astype(_BF)


def _oproj_body(o_ref, wo_ref, bo_ref, out_ref):
    acc = jnp.zeros((TS, D), _F32) + bo_ref[0, 0]
    for h in range(H):
        acc = acc + jnp.dot(o_ref[h], wo_ref[h], preferred_element_type=_F32)
    out_ref[:] = acc


def kernel(hidden_states, freqs_cos, freqs_sin, Wq, bq, Wk, bk, Wv, bv,
           Wo, bo, norm_q_w, norm_k_w):
    hs2 = hidden_states.reshape(S, D).astype(_BF)
    cos_e = freqs_cos.reshape(S, DH)[:, 0::2]
    sin_o = freqs_sin.reshape(S, DH)[:, 1::2]
    chead = jnp.repeat(cos_e, 2, axis=1)
    shead = jnp.repeat(sin_o, 2, axis=1)
    wcat = jnp.concatenate([Wq, Wk, Wv], axis=1).astype(_BF)
    bcat = jnp.concatenate([bq, bk, bv]).reshape(1, 3 * D)
    nqw = norm_q_w.reshape(1, D)
    nkw = norm_k_w.reshape(1, D)
    nqt = TS // BLKQ
    nkt = TS // BLKK
    m2 = jnp.repeat(jnp.eye(nqt, dtype=_F32), BLKQ, axis=1) / BLKQ
    m4 = jnp.repeat(jnp.eye(nkt, dtype=_F32), BLKK, axis=1) / BLKK

    q2, k2, v2, qb3, kb3 = pl.pallas_call(
        _qkv_body,
        grid=(S // TS,),
        in_specs=[
            pl.BlockSpec((TS, D), lambda i: (i, 0)),
            pl.BlockSpec((D, 3 * D), lambda i: (0, 0)),
            pl.BlockSpec((1, 3 * D), lambda i: (0, 0)),
            pl.BlockSpec((1, D), lambda i: (0, 0)),
            pl.BlockSpec((1, D), lambda i: (0, 0)),
            pl.BlockSpec((TS, DH), lambda i: (i, 0)),
            pl.BlockSpec((TS, DH), lambda i: (i, 0)),
            pl.BlockSpec((nqt, TS), lambda i: (0, 0)),
            pl.BlockSpec((nkt, TS), lambda i: (0, 0)),
        ],
        out_specs=[
            pl.BlockSpec((TS, D), lambda i: (i, 0)),
            pl.BlockSpec((TS, D), lambda i: (i, 0)),
            pl.BlockSpec((TS, D), lambda i: (i, 0)),
            pl.BlockSpec((1, nqt, D), lambda i: (i, 0, 0)),
            pl.BlockSpec((1, nkt, D), lambda i: (i, 0, 0)),
        ],
        out_shape=[
            jax.ShapeDtypeStruct((S, D), _F32),
            jax.ShapeDtypeStruct((S, D), _F32),
            jax.ShapeDtypeStruct((S, D), _BF),
            jax.ShapeDtypeStruct((S // TS, nqt, D), _F32),
            jax.ShapeDtypeStruct((S // TS, nkt, D), _F32),
        ],
    )(hs2, wcat, bcat, nqw, nkw, chead, shead, m2, m4)
    qb_all = qb3.reshape(NQ, D)
    kb_all = kb3.reshape(NK, D)

    idx = pl.pallas_call(
        _score_body,
        grid=(1,),
        in_specs=[
            pl.BlockSpec((NQ, D), lambda i: (0, 0)),
            pl.BlockSpec((NK, D), lambda i: (0, 0)),
        ],
        out_specs=pl.BlockSpec((H, NQ, KSEL), lambda i: (0, 0, 0)),
        out_shape=jax.ShapeDtypeStruct((H, NQ, KSEL), jnp.int32),
    )(qb_all, kb_all)

    o_heads = pl.pallas_call(
        _attn_body,
        grid_spec=pltpu.PrefetchScalarGridSpec(
            num_scalar_prefetch=1,
            grid=(H, 1 + NQ // QB),
            in_specs=[
                pl.BlockSpec((QB * BLKQ, DH),
                             lambda h, qg, *_: (jnp.maximum(qg - 1, 0), h)),
                pl.BlockSpec((S, DH), lambda h, qg, *_: (0, h)),
                pl.BlockSpec((S, DH), lambda h, qg, *_: (0, h)),
            ],
            out_specs=pl.BlockSpec(
                (1, QB * BLKQ, DH),
                lambda h, qg, *_: (h, jnp.maximum(qg - 1, 0), 0)),
            scratch_shapes=[
                pltpu.VMEM((NK, DH, DH), _BF),
                pltpu.VMEM((NK, 1, DH), _F32),
                pltpu.VMEM((DH, DH), _F32),
                pltpu.VMEM((1, DH), _F32),
            ],
        ),
        out_shape=jax.ShapeDtypeStruct((H, S, DH), _BF),
    )(idx, q2, k2, v2)

    wo_r = Wo.reshape(H, DH, D).astype(_BF)
    out = pl.pallas_call(
        _oproj_body,
        grid=(S // TS,),
        in_specs=[
            pl.BlockSpec((H, TS, DH), lambda i: (0, i, 0)),
            pl.BlockSpec((H, DH, D), lambda i: (0, 0, 0)),
            pl.BlockSpec((1, 1, D), lambda i: (0, 0, 0)),
        ],
        out_specs=pl.BlockSpec((TS, D), lambda i: (i, 0)),
        out_shape=jax.ShapeDtypeStruct((S, D), _F32),
    )(o_heads, wo_r, bo.reshape(1, 1, D))

    return out.reshape(B, S, D)
